# baseline (speedup 1.0000x reference)
import functools

import jax
import jax.numpy as jnp
import numpy as np
from jax.experimental import pallas as pl
from jax.experimental.pallas import tpu as pltpu

_LEVELS = np.array([8.0, 8.0, 8.0, 5.0, 5.0, 5.0], dtype=np.float32)
_NUM_Q = 8
_EPS = 1e-3
_G = 4
_NC = 4


def _compute_chunk(xb, w_in, b_in, w_out, b_out, ln_g, ln_b, consts, scales):
    hs = []
    for g in range(_G):
        hs.append(jnp.dot(w_in, xb[g], preferred_element_type=jnp.float32))
    h = jnp.stack(hs, axis=0) + b_in[None]

    mu = jnp.mean(h, axis=1, keepdims=True)
    var = jnp.mean((h - mu) ** 2, axis=1, keepdims=True)
    h = (h - mu) / jnp.sqrt(var + 1e-5) * ln_g[None] + ln_b[None]

    half_l = consts[:, 0:1][None]
    offset = consts[:, 1:2][None]
    shift = consts[:, 2:3][None]
    half_width = consts[:, 3:4][None]
    basis = consts[:, 4:5][None]

    residual = h
    qout = jnp.zeros_like(h)
    idx_rows = []
    for q in range(_NUM_Q):
        scale = scales[:, q:q + 1][None]
        z = residual / scale
        bz = jnp.tanh(z + shift) * half_l - offset
        qv = bz + (jnp.round(bz) - bz)
        codes = qv / half_width
        zhat = codes * half_width + half_width
        idx_rows.append(jnp.sum(zhat * basis, axis=1))
        emb = codes * scale
        residual = residual - emb
        qout = qout + emb

    idx = jnp.stack(idx_rows, axis=1).astype(jnp.int32)

    outs = []
    for g in range(_G):
        outs.append(jnp.dot(w_out, qout[g],
                            preferred_element_type=jnp.float32) + b_out)
    return jnp.stack(outs, axis=0), idx


def _fused_kernel(x_hbm, w_in_ref, b_in_ref, w_out_ref, b_out_ref,
                  ln_g_ref, ln_b_ref, consts_ref, scales_ref,
                  out_hbm, idx_hbm,
                  xbuf, obuf, ibuf, ins, outs, isem):
    def in_copy(i):
        return pltpu.make_async_copy(
            x_hbm.at[pl.ds(i * _G, _G)], xbuf.at[pl.ds(i * _G, _G)],
            ins.at[i])

    def out_copy(i):
        return pltpu.make_async_copy(
            obuf.at[pl.ds(i * _G, _G)], out_hbm.at[pl.ds(i * _G, _G)],
            outs.at[i])

    in_copy(0).start()
    in_copy(1).start()
    for i in range(_NC):
        in_copy(i).wait()
        if i + 2 < _NC:
            in_copy(i + 2).start()
        sl = pl.ds(i * _G, _G)
        out, idx = _compute_chunk(
            xbuf[sl], w_in_ref[...], b_in_ref[...], w_out_ref[...],
            b_out_ref[...], ln_g_ref[...], ln_b_ref[...],
            consts_ref[...], scales_ref[...])
        obuf[sl] = out
        ibuf[sl] = idx
        out_copy(i).start()
    pltpu.make_async_copy(ibuf, idx_hbm, isem).start()
    for i in range(_NC):
        out_copy(i).wait()
    pltpu.make_async_copy(ibuf, idx_hbm, isem).wait()


def kernel(x, W_in, b_in, W_out, b_out, ln_g, ln_b):
    B, D, N = x.shape
    C = W_in.shape[0]

    levels = jnp.asarray(_LEVELS)
    half_l = (levels - 1.0) * (1.0 - _EPS) / 2.0
    offset = jnp.where(jnp.mod(levels, 2.0) == 0.0, 0.5, 0.0)
    shift = jnp.arctanh(offset / half_l)
    half_width = jnp.floor(levels / 2.0)
    basis = jnp.concatenate([jnp.ones((1,), jnp.float32),
                             jnp.cumprod(levels)[:-1]])
    consts = jnp.stack([half_l, offset, shift, half_width, basis],
                       axis=1)
    scales = jnp.stack([(levels - 1.0) ** (-float(q))
                        for q in range(_NUM_Q)], axis=1)

    col = lambda v: v.reshape(-1, 1)
    vmem = functools.partial(pl.BlockSpec, memory_space=pltpu.MemorySpace.VMEM)

    out, idx_t = pl.pallas_call(
        _fused_kernel,
        in_specs=[
            pl.BlockSpec(memory_space=pltpu.MemorySpace.HBM),
            vmem(), vmem(), vmem(), vmem(), vmem(), vmem(), vmem(), vmem(),
        ],
        out_specs=[
            pl.BlockSpec(memory_space=pltpu.MemorySpace.HBM),
            pl.BlockSpec(memory_space=pltpu.MemorySpace.HBM),
        ],
        out_shape=[
            jax.ShapeDtypeStruct((B, D, N), jnp.float32),
            jax.ShapeDtypeStruct((B, _NUM_Q, N), jnp.int32),
        ],
        scratch_shapes=[
            pltpu.VMEM((B, D, N), jnp.float32),
            pltpu.VMEM((B, D, N), jnp.float32),
            pltpu.VMEM((B, _NUM_Q, N), jnp.int32),
            pltpu.SemaphoreType.DMA((_NC,)),
            pltpu.SemaphoreType.DMA((_NC,)),
            pltpu.SemaphoreType.DMA,
        ],
    )(x, W_in, col(b_in), W_out, col(b_out), col(ln_g), col(ln_b),
      consts, scales)

    return out, jnp.transpose(idx_t, (0, 2, 1))


# speedup vs baseline: 1.0204x; 1.0204x over previous
import jax
import jax.numpy as jnp
import numpy as np
from jax.experimental import pallas as pl

_LEVELS = np.array([8.0, 8.0, 8.0, 5.0, 5.0, 5.0], dtype=np.float32)
_NUM_Q = 8
_EPS = 1e-3
_G = 8


def _fused_kernel(x_ref, w_in_ref, b_in_ref, w_out_ref, b_out_ref,
                  ln_g_ref, ln_b_ref, consts_ref, scales_ref,
                  out_ref, idx_ref):
    w_in = w_in_ref[...]
    hs = []
    for g in range(_G):
        hs.append(jnp.dot(w_in, x_ref[g], preferred_element_type=jnp.float32))
    h = jnp.stack(hs, axis=0)
    h = h + b_in_ref[...][None]

    mu = jnp.mean(h, axis=1, keepdims=True)
    var = jnp.mean((h - mu) ** 2, axis=1, keepdims=True)
    h = (h - mu) / jnp.sqrt(var + 1e-5) * ln_g_ref[...][None] + ln_b_ref[...][None]

    half_l = consts_ref[:, 0:1][None]
    offset = consts_ref[:, 1:2][None]
    shift = consts_ref[:, 2:3][None]
    half_width = consts_ref[:, 3:4][None]
    basis = consts_ref[:, 4:5][None]

    residual = h
    qout = jnp.zeros_like(h)
    idx_rows = []
    for q in range(_NUM_Q):
        scale = scales_ref[:, q:q + 1][None]
        z = residual / scale
        bz = jnp.tanh(z + shift) * half_l - offset
        qv = bz + (jnp.round(bz) - bz)
        codes = qv / half_width
        zhat = codes * half_width + half_width
        idx_rows.append(jnp.sum(zhat * basis, axis=1))
        emb = codes * scale
        residual = residual - emb
        qout = qout + emb

    idx = jnp.stack(idx_rows, axis=1)
    idx_ref[...] = idx.astype(jnp.int32)

    w_out = w_out_ref[...]
    b_out = b_out_ref[...]
    for g in range(_G):
        out_ref[g] = jnp.dot(w_out, qout[g],
                             preferred_element_type=jnp.float32) + b_out


def kernel(x, W_in, b_in, W_out, b_out, ln_g, ln_b):
    B, D, N = x.shape
    C = W_in.shape[0]

    levels = jnp.asarray(_LEVELS)
    half_l = (levels - 1.0) * (1.0 - _EPS) / 2.0
    offset = jnp.where(jnp.mod(levels, 2.0) == 0.0, 0.5, 0.0)
    shift = jnp.arctanh(offset / half_l)
    half_width = jnp.floor(levels / 2.0)
    basis = jnp.concatenate([jnp.ones((1,), jnp.float32),
                             jnp.cumprod(levels)[:-1]])
    consts = jnp.stack([half_l, offset, shift, half_width, basis],
                       axis=1)
    scales = jnp.stack([(levels - 1.0) ** (-float(q))
                        for q in range(_NUM_Q)], axis=1)

    col = lambda v: v.reshape(-1, 1)
    nb = B // _G

    out, idx_t = pl.pallas_call(
        _fused_kernel,
        grid=(nb,),
        in_specs=[
            pl.BlockSpec((_G, D, N), lambda b: (b, 0, 0)),
            pl.BlockSpec((C, D), lambda b: (0, 0)),
            pl.BlockSpec((C, 1), lambda b: (0, 0)),
            pl.BlockSpec((D, C), lambda b: (0, 0)),
            pl.BlockSpec((D, 1), lambda b: (0, 0)),
            pl.BlockSpec((C, 1), lambda b: (0, 0)),
            pl.BlockSpec((C, 1), lambda b: (0, 0)),
            pl.BlockSpec((C, 5), lambda b: (0, 0)),
            pl.BlockSpec((C, _NUM_Q), lambda b: (0, 0)),
        ],
        out_specs=[
            pl.BlockSpec((_G, D, N), lambda b: (b, 0, 0)),
            pl.BlockSpec((_G, _NUM_Q, N), lambda b: (b, 0, 0)),
        ],
        out_shape=[
            jax.ShapeDtypeStruct((B, D, N), jnp.float32),
            jax.ShapeDtypeStruct((B, _NUM_Q, N), jnp.int32),
        ],
    )(x, W_in, col(b_in), W_out, col(b_out), col(ln_g), col(ln_b),
      consts, scales)

    return out, jnp.transpose(idx_t, (0, 2, 1))
